# Initial kernel scaffold; baseline (speedup 1.0000x reference)
import jax
import jax.numpy as jnp
from jax.experimental import pallas as pl


def kernel(patches):
    raise NotImplementedError("write your pallas kernel here")


# speedup vs baseline: 15.5775x; 15.5775x over previous
import functools

import jax
import jax.numpy as jnp
from jax import lax
from jax.experimental import pallas as pl
from jax.experimental.pallas import tpu as pltpu
from jax.experimental.pallas import tpu_sc as plsc

T, B, C = 196, 256, 768
RATIO = 0.75
REMAIN = int(T * (1 - RATIO))
ROWS = T * B
NC, NS, L = 2, 16, 16
NW = NC * NS
RPW = ROWS // NW
CHUNK = 112
NCHUNK = RPW // CHUNK
NBLK = RPW // L
BPC = CHUNK // L
ROWS1 = REMAIN * B
W1 = ROWS1 // RPW

_mesh = plsc.VectorSubcoreMesh(
    core_axis_name="c", subcore_axis_name="s", num_cores=NC, num_subcores=NS
)


@functools.partial(
    pl.kernel,
    mesh=_mesh,
    out_type=(
        jax.ShapeDtypeStruct((ROWS1, C), jnp.float32),
        jax.ShapeDtypeStruct((ROWS - ROWS1, C), jnp.float32),
        jax.ShapeDtypeStruct((ROWS,), jnp.int32),
    ),
    scratch_types=[
        pltpu.VMEM((RPW,), jnp.int32),
        pltpu.VMEM((NCHUNK, CHUNK), jnp.int32),
        pltpu.VMEM((NCHUNK, CHUNK), jnp.int32),
        pltpu.VMEM((CHUNK, C), jnp.float32),
        pltpu.SemaphoreType.DMA,
    ],
)
def _shuffle_sc(fwd_flat_hbm, table_hbm, out1_hbm, out2_hbm, bwd_hbm,
                fwd_v, idx_v, tval_v, rows_v, sem):
    w = lax.axis_index("s") * NC + lax.axis_index("c")
    base = w * RPW
    lane = lax.iota(jnp.int32, L)

    pltpu.sync_copy(fwd_flat_hbm.at[pl.ds(base, RPW)], fwd_v)

    def build(j, carry):
        c = lax.div(j, BPC)
        k = lax.rem(j, BPC)
        off = pl.multiple_of(j * L, 8)
        koff = pl.multiple_of(k * L, 8)
        f = fwd_v[pl.ds(off, L)]
        r0 = base + j * L
        idx_v[c, pl.ds(koff, L)] = f * B + (lax.rem(r0, B) + lane)
        tval_v[c, pl.ds(koff, L)] = jnp.zeros((L,), jnp.int32) + lax.div(r0, B)
        return carry

    lax.fori_loop(0, NBLK, build, 0)

    def bscat(c, carry):
        pltpu.async_copy(tval_v.at[c], bwd_hbm.at[idx_v.at[c]], sem).wait()
        return carry

    lax.fori_loop(0, NCHUNK, bscat, 0)

    def run(out_ref, local_base):
        def step(c, carry):
            pltpu.async_copy(table_hbm.at[idx_v.at[c]], rows_v, sem).wait()
            pltpu.sync_copy(
                rows_v, out_ref.at[pl.ds(local_base + c * CHUNK, CHUNK), :]
            )
            return carry

        lax.fori_loop(0, NCHUNK, step, 0)

    @pl.when(w < W1)
    def _():
        run(out1_hbm, base)

    @pl.when(w >= W1)
    def _():
        run(out2_hbm, base - ROWS1)


def _forward_indexes():
    keys = jax.random.split(jax.random.key(1), B)
    fwd = jax.vmap(lambda k: jax.random.permutation(k, T))(keys).T
    return fwd.astype(jnp.int32)


def kernel(patches):
    fwd = _forward_indexes()
    table = patches.reshape(ROWS, C)
    out1, out2, bwd = _shuffle_sc(fwd.reshape(ROWS), table)
    patches_1 = out1.reshape(REMAIN, B, C)
    patches_2 = out2.reshape(T - REMAIN, B, C)
    return (patches_1, patches_2,
            fwd.astype(jnp.int64), bwd.reshape(T, B).astype(jnp.int64))


# speedup vs baseline: 16.2040x; 1.0402x over previous
import functools

import jax
import jax.numpy as jnp
from jax import lax
from jax.experimental import pallas as pl
from jax.experimental.pallas import tpu as pltpu
from jax.experimental.pallas import tpu_sc as plsc

T, B, C = 196, 256, 768
RATIO = 0.75
REMAIN = int(T * (1 - RATIO))
ROWS = T * B
NC, NS, L = 2, 16, 16
NW = NC * NS
RPW = ROWS // NW
CHUNK = 112
NCHUNK = RPW // CHUNK
HCHUNK = CHUNK // 2
NHALF = 2 * NCHUNK
NBLK = RPW // L
BPC = CHUNK // L
ROWS1 = REMAIN * B
W1 = ROWS1 // RPW

_mesh = plsc.VectorSubcoreMesh(
    core_axis_name="c", subcore_axis_name="s", num_cores=NC, num_subcores=NS
)


@functools.partial(
    pl.kernel,
    mesh=_mesh,
    out_type=(
        jax.ShapeDtypeStruct((ROWS1, C), jnp.float32),
        jax.ShapeDtypeStruct((ROWS - ROWS1, C), jnp.float32),
        jax.ShapeDtypeStruct((ROWS,), jnp.int32),
    ),
    scratch_types=[
        pltpu.VMEM((RPW,), jnp.int32),
        pltpu.VMEM((RPW,), jnp.int32),
        pltpu.VMEM((NCHUNK, CHUNK), jnp.int32),
        pltpu.VMEM((NCHUNK, CHUNK), jnp.int32),
        pltpu.VMEM((HCHUNK, C), jnp.float32),
        pltpu.VMEM((HCHUNK, C), jnp.float32),
        pltpu.SemaphoreType.DMA,
        pltpu.SemaphoreType.DMA,
        pltpu.SemaphoreType.DMA,
    ],
)
def _shuffle_sc(fwd_flat_hbm, table_hbm, out1_hbm, out2_hbm, bwd_hbm,
                fwd_v, idxf_v, idx_v, tval_v, rows0_v, rows1_v,
                gsem0, gsem1, bsem):
    w = lax.axis_index("s") * NC + lax.axis_index("c")
    base = w * RPW
    lane = lax.iota(jnp.int32, L)

    pltpu.sync_copy(fwd_flat_hbm.at[pl.ds(base, RPW)], fwd_v)

    def build(j, carry):
        c = lax.div(j, BPC)
        k = lax.rem(j, BPC)
        off = pl.multiple_of(j * L, 8)
        koff = pl.multiple_of(k * L, 8)
        f = fwd_v[pl.ds(off, L)]
        r0 = base + j * L
        iv = f * B + (lax.rem(r0, B) + lane)
        idxf_v[pl.ds(off, L)] = iv
        idx_v[c, pl.ds(koff, L)] = iv
        tval_v[c, pl.ds(koff, L)] = jnp.zeros((L,), jnp.int32) + lax.div(r0, B)
        return carry

    lax.fori_loop(0, NBLK, build, 0)

    for c in range(NCHUNK):
        pltpu.async_copy(tval_v.at[c], bwd_hbm.at[idx_v.at[c]], bsem)

    bufs = ((rows0_v, gsem0), (rows1_v, gsem1))

    def _ghalf(h, rows, gsem):
        iref = idxf_v.at[pl.ds(h * HCHUNK, HCHUNK)]
        return pltpu.make_async_copy(table_hbm.at[iref], rows, gsem)

    def run(out_ref, local_base):
        _ghalf(0, rows0_v, gsem0).start()
        _ghalf(1, rows1_v, gsem1).start()

        def step(i, carry):
            for b, (rows, gsem) in enumerate(bufs):
                h = 2 * i + b
                _ghalf(h, rows, gsem).wait()
                pltpu.sync_copy(
                    rows, out_ref.at[pl.ds(local_base + h * HCHUNK, HCHUNK), :]
                )

                @pl.when(h + 2 < NHALF)
                def _():
                    _ghalf(h + 2, rows, gsem).start()

            return carry

        lax.fori_loop(0, NHALF // 2, step, 0)

    @pl.when(w < W1)
    def _():
        run(out1_hbm, base)

    @pl.when(w >= W1)
    def _():
        run(out2_hbm, base - ROWS1)

    for c in range(NCHUNK):
        pltpu.make_async_copy(tval_v.at[c], bwd_hbm.at[idx_v.at[c]], bsem).wait()


def _forward_indexes():
    keys = jax.random.split(jax.random.key(1), B)
    fwd = jax.vmap(lambda k: jax.random.permutation(k, T))(keys).T
    return fwd.astype(jnp.int32)


def kernel(patches):
    fwd = _forward_indexes()
    table = patches.reshape(ROWS, C)
    out1, out2, bwd = _shuffle_sc(fwd.reshape(ROWS), table)
    patches_1 = out1.reshape(REMAIN, B, C)
    patches_2 = out2.reshape(T - REMAIN, B, C)
    return (patches_1, patches_2,
            fwd.astype(jnp.int64), bwd.reshape(T, B).astype(jnp.int64))


# speedup vs baseline: 16.4857x; 1.0174x over previous
import functools

import jax
import jax.numpy as jnp
from jax import lax
from jax.experimental import pallas as pl
from jax.experimental.pallas import tpu as pltpu
from jax.experimental.pallas import tpu_sc as plsc

T, B, C = 196, 256, 768
RATIO = 0.75
REMAIN = int(T * (1 - RATIO))
ROWS = T * B
NC, NS, L = 2, 16, 16
NW = NC * NS
RPW = ROWS // NW
CHUNK = 112
NCHUNK = RPW // CHUNK
GCH = 16
NGCH = RPW // GCH
NB = 7
KLA = 4
NROUND = NGCH // NB
NBLK = RPW // L
BPC = CHUNK // L
ROWS1 = REMAIN * B
W1 = ROWS1 // RPW

_mesh = plsc.VectorSubcoreMesh(
    core_axis_name="c", subcore_axis_name="s", num_cores=NC, num_subcores=NS
)


@functools.partial(
    pl.kernel,
    mesh=_mesh,
    out_type=(
        jax.ShapeDtypeStruct((ROWS1, C), jnp.float32),
        jax.ShapeDtypeStruct((ROWS - ROWS1, C), jnp.float32),
        jax.ShapeDtypeStruct((ROWS,), jnp.int32),
    ),
    scratch_types=[
        pltpu.VMEM((RPW,), jnp.int32),
        pltpu.VMEM((RPW,), jnp.int32),
        pltpu.VMEM((NCHUNK, CHUNK), jnp.int32),
        pltpu.VMEM((NCHUNK, CHUNK), jnp.int32),
        [pltpu.VMEM((GCH, C), jnp.float32)] * NB,
        [pltpu.SemaphoreType.DMA] * NB,
        [pltpu.SemaphoreType.DMA] * NB,
        pltpu.SemaphoreType.DMA,
    ],
)
def _shuffle_sc(fwd_flat_hbm, table_hbm, out1_hbm, out2_hbm, bwd_hbm,
                fwd_v, idxf_v, idx_v, tval_v, rows, gsem, ssem, bsem):
    w = lax.axis_index("s") * NC + lax.axis_index("c")
    base = w * RPW
    lane = lax.iota(jnp.int32, L)

    pltpu.sync_copy(fwd_flat_hbm.at[pl.ds(base, RPW)], fwd_v)

    def build(j, carry):
        c = lax.div(j, BPC)
        k = lax.rem(j, BPC)
        off = pl.multiple_of(j * L, 8)
        koff = pl.multiple_of(k * L, 8)
        f = fwd_v[pl.ds(off, L)]
        r0 = base + j * L
        iv = f * B + (lax.rem(r0, B) + lane)
        idxf_v[pl.ds(off, L)] = iv
        idx_v[c, pl.ds(koff, L)] = iv
        tval_v[c, pl.ds(koff, L)] = jnp.zeros((L,), jnp.int32) + lax.div(r0, B)
        return carry

    lax.fori_loop(0, NBLK, build, 0)

    for c in range(NCHUNK):
        pltpu.async_copy(tval_v.at[c], bwd_hbm.at[idx_v.at[c]], bsem)

    def _gath(h, s):
        iref = idxf_v.at[pl.ds(h * GCH, GCH)]
        return pltpu.make_async_copy(table_hbm.at[iref], rows[s], gsem[s])

    def run(out_ref, local_base):
        def _stor(h, s):
            dst = out_ref.at[pl.ds(local_base + h * GCH, GCH), :]
            return pltpu.make_async_copy(rows[s], dst, ssem[s])

        for h in range(KLA):
            _gath(h, h).start()

        def round_body(r, carry):
            for s in range(NB):
                h = r * NB + s
                _gath(h, s).wait()
                _stor(h, s).start()
                s2 = (s + KLA) % NB
                h2 = h + KLA

                @pl.when(h2 < NGCH)
                def _():
                    @pl.when(h2 >= NB)
                    def _():
                        _stor(h2 - NB, s2).wait()
                    _gath(h2, s2).start()

            return carry

        lax.fori_loop(0, NROUND, round_body, 0)
        for s in range(NB):
            _stor(NGCH - NB + s, s).wait()

    @pl.when(w < W1)
    def _():
        run(out1_hbm, base)

    @pl.when(w >= W1)
    def _():
        run(out2_hbm, base - ROWS1)

    for c in range(NCHUNK):
        pltpu.make_async_copy(tval_v.at[c], bwd_hbm.at[idx_v.at[c]], bsem).wait()


def _forward_indexes():
    keys = jax.random.split(jax.random.key(1), B)
    fwd = jax.vmap(lambda k: jax.random.permutation(k, T))(keys).T
    return fwd.astype(jnp.int32)


def kernel(patches):
    fwd = _forward_indexes()
    table = patches.reshape(ROWS, C)
    out1, out2, bwd = _shuffle_sc(fwd.reshape(ROWS), table)
    patches_1 = out1.reshape(REMAIN, B, C)
    patches_2 = out2.reshape(T - REMAIN, B, C)
    return (patches_1, patches_2,
            fwd.astype(jnp.int64), bwd.reshape(T, B).astype(jnp.int64))
